# baseline (speedup 1.0000x reference)
import jax
import jax.numpy as jnp
from jax import lax
from jax.experimental import pallas as pl
from jax.experimental.pallas import tpu as pltpu
from jax.experimental.pallas import tpu_sc as plsc

N = 10000
E = 320000
NW = 32
B = 128
NB = 80
EPT = B * NB
E_PAD = NW * EPT
N_ACC = 10112
STRIPE = N_ACC // 16
R = 1000
NBW = 2 * NB

_F32 = jnp.float32


def _sc_aggregate(z, edges_r, zeros_acc):
    D = z.shape[1]
    mesh = plsc.VectorSubcoreMesh(
        core_axis_name="c", subcore_axis_name="s", num_cores=2, num_subcores=16
    )
    out_type = jax.ShapeDtypeStruct((N_ACC, D), _F32)
    scratch = [
        pltpu.VMEM((4, 2, B), jnp.int32),
        pltpu.VMEM((2, B, D), _F32),
        pltpu.VMEM_SHARED((N_ACC, D), _F32),
        pltpu.SemaphoreType.DMA,
        pltpu.SemaphoreType.DMA,
        pltpu.SemaphoreType.DMA,
        pltpu.SemaphoreType.DMA,
        pltpu.SemaphoreType.DMA,
        pltpu.SemaphoreType.DMA,
    ]

    def body(z_hbm, edg_hbm, zer_hbm, s0_out,
             idx_v, rows_v, acc_sh, si0, si1, si2, si3, sg0, sg1):
        c = lax.axis_index("c")
        s = lax.axis_index("s")

        @pl.when(c == 0)
        def _work():
            base = s * STRIPE
            boff = s * NBW
            sis = (si0, si1, si2, si3)
            sgs = (sg0, sg1)

            def idx_cp(g, k):
                return pltpu.make_async_copy(edg_hbm.at[boff + g], idx_v.at[k], sis[k])

            def gat_cp(g, k):
                return pltpu.make_async_copy(
                    z_hbm.at[idx_v.at[k, 0]], rows_v.at[k % 2], sgs[k % 2])

            for k in range(4):
                idx_cp(k, k).start()
            pltpu.sync_copy(zer_hbm.at[pl.ds(base, STRIPE)], acc_sh.at[pl.ds(base, STRIPE)])
            idx_cp(0, 0).wait()
            gat_cp(0, 0).start()
            idx_cp(1, 1).wait()
            gat_cp(1, 1).start()
            plsc.subcore_barrier()

            def loop(i, cr):
                for j in range(4):
                    g = i * 4 + j
                    gat_cp(g, j).wait()
                    pltpu.sync_copy(rows_v.at[j % 2], acc_sh.at[idx_v.at[j, 1]], add=True)

                    @pl.when(g + 4 < NBW)
                    def _():
                        idx_cp(g + 4, j).start()

                    @pl.when(g + 2 < NBW)
                    def _():
                        idx_cp(g + 2, (j + 2) % 4).wait()
                        gat_cp(g + 2, (j + 2) % 4).start()
                return cr

            lax.fori_loop(0, NBW // 4, loop, 0)
            plsc.subcore_barrier()
            pltpu.sync_copy(acc_sh.at[pl.ds(base, STRIPE)], s0_out.at[pl.ds(base, STRIPE)])

    return pl.kernel(
        body, out_type=out_type, mesh=mesh, scratch_types=scratch,
        compiler_params=pltpu.CompilerParams(use_tc_tiling_on_sc=False),
    )(z, edges_r, zeros_acc)


def _mm_first(x_ref, ws_ref, wn_ref, b_ref, y_ref, z_ref):
    x = x_ref[...]
    y_ref[...] = jnp.dot(x, ws_ref[...], preferred_element_type=_F32) + b_ref[0:1, :]
    z_ref[:, 0:128] = jnp.dot(x, wn_ref[...], preferred_element_type=_F32)
    z_ref[:, 128:144] = jnp.ones((R, 16), _F32)


def _mm_mid(y_in_ref, s0_ref, d0_ref, ws_ref, wn_ref, b_ref, y_ref, z_ref):
    deg = d0_ref[:, 128:129]
    invd = 1.0 / jnp.maximum(deg, 1.0)
    h = y_in_ref[...] + s0_ref[...] * invd
    h = jnp.maximum(h, 0.0)
    y_ref[...] = jnp.dot(h, ws_ref[...], preferred_element_type=_F32) + b_ref[0:1, :]
    z_ref[...] = jnp.dot(h, wn_ref[...], preferred_element_type=_F32)


def _mm_final(y_in_ref, s0_ref, d0_ref, o_ref):
    deg = d0_ref[:, 128:129]
    invd = 1.0 / jnp.maximum(deg, 1.0)
    h = y_in_ref[...] + s0_ref[...] * invd

    @pl.when(pl.program_id(0) == 0)
    def _():
        o_ref[...] = jnp.zeros_like(o_ref)

    o_ref[0:1, :] += jnp.sum(h, axis=0, keepdims=True) * (1.0 / N)


def _tc_first(x, ws, wn, b8):
    din, d = ws.shape
    return pl.pallas_call(
        _mm_first,
        grid=(N // R,),
        in_specs=[
            pl.BlockSpec((R, din), lambda i: (i, 0)),
            pl.BlockSpec((din, d), lambda i: (0, 0)),
            pl.BlockSpec((din, d), lambda i: (0, 0)),
            pl.BlockSpec((8, d), lambda i: (0, 0)),
        ],
        out_specs=[
            pl.BlockSpec((R, d), lambda i: (i, 0)),
            pl.BlockSpec((R, d + 16), lambda i: (i, 0)),
        ],
        out_shape=[
            jax.ShapeDtypeStruct((N, d), _F32),
            jax.ShapeDtypeStruct((N, d + 16), _F32),
        ],
    )(x, ws, wn, b8)


def _tc_mid(y, s0, dg0, ws, wn, b8):
    din, d = ws.shape
    dw = dg0.shape[1]
    return pl.pallas_call(
        _mm_mid,
        grid=(N // R,),
        in_specs=[
            pl.BlockSpec((R, din), lambda i: (i, 0)),
            pl.BlockSpec((R, din), lambda i: (i, 0)),
            pl.BlockSpec((R, dw), lambda i: (i, 0)),
            pl.BlockSpec((din, d), lambda i: (0, 0)),
            pl.BlockSpec((din, d), lambda i: (0, 0)),
            pl.BlockSpec((8, d), lambda i: (0, 0)),
        ],
        out_specs=[pl.BlockSpec((R, d), lambda i: (i, 0))] * 2,
        out_shape=[jax.ShapeDtypeStruct((N, d), _F32)] * 2,
    )(y, s0, dg0, ws, wn, b8)


def _tc_final(y, s0, dg0):
    d = y.shape[1]
    dw = dg0.shape[1]
    return pl.pallas_call(
        _mm_final,
        grid=(N // R,),
        in_specs=[
            pl.BlockSpec((R, d), lambda i: (i, 0)),
            pl.BlockSpec((R, d), lambda i: (i, 0)),
            pl.BlockSpec((R, dw), lambda i: (i, 0)),
        ],
        out_specs=pl.BlockSpec((8, d), lambda i: (0, 0)),
        out_shape=jax.ShapeDtypeStruct((8, d), _F32),
    )(y, s0, dg0)


def _pad2(w, din_p, d_p):
    return jnp.pad(w.astype(_F32), ((0, din_p - w.shape[0]), (0, d_p - w.shape[1])))


def _pad_b(b, d_p):
    bp = jnp.pad(b.astype(_F32), (0, d_p - b.shape[0]))
    return jnp.broadcast_to(bp[None, :], (8, d_p))


def kernel(in_feat, edge_index, W_self1, W_neigh1, b1, W_self2, W_neigh2, b2,
           W_self3, W_neigh3, b3, W_self4, W_neigh4, b4):
    d1, d2, d3, d4 = 128, 128, 112, 48

    src = edge_index[0].astype(jnp.int32)
    dst = edge_index[1].astype(jnp.int32)
    pad = E_PAD - E
    src_r = jnp.concatenate([src, jnp.zeros((pad,), jnp.int32)]).reshape(NW * NB, 1, B)
    dst_r = jnp.concatenate([dst, jnp.full((pad,), N, jnp.int32)]).reshape(NW * NB, 1, B)
    edges_r = jnp.concatenate([src_r, dst_r], axis=1)

    ws1, wn1, bb1 = _pad2(W_self1, 128, d1), _pad2(W_neigh1, 128, d1), _pad_b(b1, d1)
    ws2, wn2, bb2 = _pad2(W_self2, d1, d2), _pad2(W_neigh2, d1, d2), _pad_b(b2, d2)
    ws3, wn3, bb3 = _pad2(W_self3, d2, d3), _pad2(W_neigh3, d2, d3), _pad_b(b3, d3)
    ws4, wn4, bb4 = _pad2(W_self4, d3, d4), _pad2(W_neigh4, d3, d4), _pad_b(b4, d4)

    z1_14 = jnp.zeros((N_ACC, d1 + 16), _F32)
    z128 = jnp.zeros((N_ACC, d2), _F32)
    z112 = jnp.zeros((N_ACC, d3), _F32)
    z48 = jnp.zeros((N_ACC, d4), _F32)

    y1, z1 = _tc_first(in_feat.astype(_F32), ws1, wn1, bb1)
    sa = _sc_aggregate(z1, edges_r, z1_14)
    y2, z2 = _tc_mid(y1, sa, sa, ws2, wn2, bb2)
    sb = _sc_aggregate(z2, edges_r, z128)
    y3, z3 = _tc_mid(y2, sb, sa, ws3, wn3, bb3)
    sc = _sc_aggregate(z3, edges_r, z112)
    y4, z4 = _tc_mid(y3, sc, sa, ws4, wn4, bb4)
    sd = _sc_aggregate(z4, edges_r, z48)
    out = _tc_final(y4, sd, sa)
    return out[0, :40]


# speedup vs baseline: 3.4835x; 3.4835x over previous
import jax
import jax.numpy as jnp
from jax import lax
from jax.experimental import pallas as pl
from jax.experimental.pallas import tpu as pltpu
from jax.experimental.pallas import tpu_sc as plsc

N = 10000
E = 320000
NW = 32
B = 128
NB = 80
EPT = B * NB
E_PAD = NW * EPT
N_ACC = 10112
STRIPE = N_ACC // 16
R = 1000
NB0 = 80
NB1 = 160 - NB0

_F32 = jnp.float32


def _sc_aggregate(z, edges_r, zeros_acc):
    D = z.shape[1]
    mesh = plsc.VectorSubcoreMesh(
        core_axis_name="c", subcore_axis_name="s", num_cores=2, num_subcores=16
    )
    out_type = (
        jax.ShapeDtypeStruct((N_ACC, D), _F32),
        jax.ShapeDtypeStruct((N_ACC, D), _F32),
    )
    scratch = [
        pltpu.VMEM((4, 2, B), jnp.int32),
        pltpu.VMEM((2, B, D), _F32),
        pltpu.VMEM_SHARED((N_ACC, D), _F32),
        pltpu.SemaphoreType.DMA,
        pltpu.SemaphoreType.DMA,
        pltpu.SemaphoreType.DMA,
        pltpu.SemaphoreType.DMA,
        pltpu.SemaphoreType.DMA,
        pltpu.SemaphoreType.DMA,
    ]

    def body(z_hbm, edg_hbm, zer_hbm, s0_out, s1_out,
             idx_v, rows_v, acc_sh, si0, si1, si2, si3, sg0, sg1):
        c = lax.axis_index("c")
        s = lax.axis_index("s")
        base = s * STRIPE
        nb = jnp.where(c == 0, NB0, NB1)
        boff = jnp.where(c == 0, s * NB0, 16 * NB0 + s * NB1)
        sis = (si0, si1, si2, si3)
        sgs = (sg0, sg1)

        def idx_cp(g, k):
            return pltpu.make_async_copy(edg_hbm.at[boff + g], idx_v.at[k], sis[k])

        def gat_cp(g, k):
            return pltpu.make_async_copy(
                z_hbm.at[idx_v.at[k, 0]], rows_v.at[k % 2], sgs[k % 2])

        for k in range(4):
            idx_cp(k, k).start()
        pltpu.sync_copy(zer_hbm.at[pl.ds(base, STRIPE)], acc_sh.at[pl.ds(base, STRIPE)])
        idx_cp(0, 0).wait()
        gat_cp(0, 0).start()
        idx_cp(1, 1).wait()
        gat_cp(1, 1).start()
        plsc.subcore_barrier()

        def loop(i, cr):
            for j in range(4):
                g = i * 4 + j
                gat_cp(g, j).wait()
                pltpu.sync_copy(rows_v.at[j % 2], acc_sh.at[idx_v.at[j, 1]], add=True)

                @pl.when(g + 4 < nb)
                def _():
                    idx_cp(g + 4, j).start()

                @pl.when(g + 2 < nb)
                def _():
                    idx_cp(g + 2, (j + 2) % 4).wait()
                    gat_cp(g + 2, (j + 2) % 4).start()
            return cr

        lax.fori_loop(0, nb // 4, loop, 0)
        plsc.subcore_barrier()

        @pl.when(c == 0)
        def _():
            pltpu.sync_copy(acc_sh.at[pl.ds(base, STRIPE)], s0_out.at[pl.ds(base, STRIPE)])

        @pl.when(c == 1)
        def _():
            pltpu.sync_copy(acc_sh.at[pl.ds(base, STRIPE)], s1_out.at[pl.ds(base, STRIPE)])

    return pl.kernel(
        body, out_type=out_type, mesh=mesh, scratch_types=scratch,
        compiler_params=pltpu.CompilerParams(use_tc_tiling_on_sc=False),
    )(z, edges_r, zeros_acc)


def _mm_first(x_ref, ws_ref, wn_ref, b_ref, y_ref, z_ref):
    x = x_ref[...]
    y_ref[...] = jnp.dot(x, ws_ref[...], preferred_element_type=_F32) + b_ref[0:1, :]
    z_ref[:, 0:128] = jnp.dot(x, wn_ref[...], preferred_element_type=_F32)
    z_ref[:, 128:144] = jnp.ones((R, 16), _F32)


def _mm_mid(y_in_ref, s0_ref, s1_ref, d0_ref, d1_ref, ws_ref, wn_ref, b_ref, y_ref, z_ref):
    deg = d0_ref[:, 128:129] + d1_ref[:, 128:129]
    invd = 1.0 / jnp.maximum(deg, 1.0)
    h = y_in_ref[...] + (s0_ref[...] + s1_ref[...]) * invd
    h = jnp.maximum(h, 0.0)
    y_ref[...] = jnp.dot(h, ws_ref[...], preferred_element_type=_F32) + b_ref[0:1, :]
    z_ref[...] = jnp.dot(h, wn_ref[...], preferred_element_type=_F32)


def _mm_final(y_in_ref, s0_ref, s1_ref, d0_ref, d1_ref, o_ref):
    deg = d0_ref[:, 128:129] + d1_ref[:, 128:129]
    invd = 1.0 / jnp.maximum(deg, 1.0)
    h = y_in_ref[...] + (s0_ref[...] + s1_ref[...]) * invd

    @pl.when(pl.program_id(0) == 0)
    def _():
        o_ref[...] = jnp.zeros_like(o_ref)

    o_ref[0:1, :] += jnp.sum(h, axis=0, keepdims=True) * (1.0 / N)


def _tc_first(x, ws, wn, b8):
    din, d = ws.shape
    return pl.pallas_call(
        _mm_first,
        grid=(N // R,),
        in_specs=[
            pl.BlockSpec((R, din), lambda i: (i, 0)),
            pl.BlockSpec((din, d), lambda i: (0, 0)),
            pl.BlockSpec((din, d), lambda i: (0, 0)),
            pl.BlockSpec((8, d), lambda i: (0, 0)),
        ],
        out_specs=[
            pl.BlockSpec((R, d), lambda i: (i, 0)),
            pl.BlockSpec((R, d + 16), lambda i: (i, 0)),
        ],
        out_shape=[
            jax.ShapeDtypeStruct((N, d), _F32),
            jax.ShapeDtypeStruct((N, d + 16), _F32),
        ],
    )(x, ws, wn, b8)


def _tc_mid(y, s0, s1, dg0, dg1, ws, wn, b8):
    din, d = ws.shape
    dw = dg0.shape[1]
    return pl.pallas_call(
        _mm_mid,
        grid=(N // R,),
        in_specs=[
            pl.BlockSpec((R, din), lambda i: (i, 0)),
            pl.BlockSpec((R, din), lambda i: (i, 0)),
            pl.BlockSpec((R, din), lambda i: (i, 0)),
            pl.BlockSpec((R, dw), lambda i: (i, 0)),
            pl.BlockSpec((R, dw), lambda i: (i, 0)),
            pl.BlockSpec((din, d), lambda i: (0, 0)),
            pl.BlockSpec((din, d), lambda i: (0, 0)),
            pl.BlockSpec((8, d), lambda i: (0, 0)),
        ],
        out_specs=[pl.BlockSpec((R, d), lambda i: (i, 0))] * 2,
        out_shape=[jax.ShapeDtypeStruct((N, d), _F32)] * 2,
    )(y, s0, s1, dg0, dg1, ws, wn, b8)


def _tc_final(y, s0, s1, dg0, dg1):
    d = y.shape[1]
    dw = dg0.shape[1]
    return pl.pallas_call(
        _mm_final,
        grid=(N // R,),
        in_specs=[
            pl.BlockSpec((R, d), lambda i: (i, 0)),
            pl.BlockSpec((R, d), lambda i: (i, 0)),
            pl.BlockSpec((R, d), lambda i: (i, 0)),
            pl.BlockSpec((R, dw), lambda i: (i, 0)),
            pl.BlockSpec((R, dw), lambda i: (i, 0)),
        ],
        out_specs=pl.BlockSpec((8, d), lambda i: (0, 0)),
        out_shape=jax.ShapeDtypeStruct((8, d), _F32),
    )(y, s0, s1, dg0, dg1)


def _pad2(w, din_p, d_p):
    return jnp.pad(w.astype(_F32), ((0, din_p - w.shape[0]), (0, d_p - w.shape[1])))


def _pad_b(b, d_p):
    bp = jnp.pad(b.astype(_F32), (0, d_p - b.shape[0]))
    return jnp.broadcast_to(bp[None, :], (8, d_p))


def kernel(in_feat, edge_index, W_self1, W_neigh1, b1, W_self2, W_neigh2, b2,
           W_self3, W_neigh3, b3, W_self4, W_neigh4, b4):
    d1, d2, d3, d4 = 128, 128, 112, 48

    src = edge_index[0].astype(jnp.int32)
    dst = edge_index[1].astype(jnp.int32)
    pad = E_PAD - E
    pidx = jnp.arange(pad, dtype=jnp.int32)
    src_r = jnp.concatenate([src, pidx % N]).reshape(NW * NB, 1, B)
    dst_r = jnp.concatenate([dst, N + pidx % (N_ACC - N)]).reshape(NW * NB, 1, B)
    edges_r = jnp.concatenate([src_r, dst_r], axis=1)

    ws1, wn1, bb1 = _pad2(W_self1, 128, d1), _pad2(W_neigh1, 128, d1), _pad_b(b1, d1)
    ws2, wn2, bb2 = _pad2(W_self2, d1, d2), _pad2(W_neigh2, d1, d2), _pad_b(b2, d2)
    ws3, wn3, bb3 = _pad2(W_self3, d2, d3), _pad2(W_neigh3, d2, d3), _pad_b(b3, d3)
    ws4, wn4, bb4 = _pad2(W_self4, d3, d4), _pad2(W_neigh4, d3, d4), _pad_b(b4, d4)

    z1_14 = jnp.zeros((N_ACC, d1 + 16), _F32)
    z128 = jnp.zeros((N_ACC, d2), _F32)
    z112 = jnp.zeros((N_ACC, d3), _F32)
    z48 = jnp.zeros((N_ACC, d4), _F32)

    y1, z1 = _tc_first(in_feat.astype(_F32), ws1, wn1, bb1)
    sa0, sa1 = _sc_aggregate(z1, edges_r, z1_14)
    y2, z2 = _tc_mid(y1, sa0, sa1, sa0, sa1, ws2, wn2, bb2)
    sb0, sb1 = _sc_aggregate(z2, edges_r, z128)
    y3, z3 = _tc_mid(y2, sb0, sb1, sa0, sa1, ws3, wn3, bb3)
    sc0, sc1 = _sc_aggregate(z3, edges_r, z112)
    y4, z4 = _tc_mid(y3, sc0, sc1, sa0, sa1, ws4, wn4, bb4)
    sd0, sd1 = _sc_aggregate(z4, edges_r, z48)
    out = _tc_final(y4, sd0, sd1, sa0, sa1)
    return out[0, :40]
